# Initial kernel scaffold; baseline (speedup 1.0000x reference)
import jax
import jax.numpy as jnp
from jax.experimental import pallas as pl


def kernel(combined, Wg, We1, be1, Weh, beh, Weo, beo, Ws1, bs1, Wsh, bsh, Wso, bso):
    raise NotImplementedError("write your pallas kernel here")


# speedup vs baseline: 1.4072x; 1.4072x over previous
import functools

import jax
import jax.numpy as jnp
import numpy as np
from jax.experimental import pallas as pl
from jax.experimental.pallas import tpu as pltpu

N = 16384
D = 256
H = 128
O = 128
E = 4
BN_S = 1.0 / np.sqrt(1.0 + 1e-5)

TN = 512


def _dot_t(a, b):
    return jax.lax.dot_general(a, b, (((1,), (1,)), ((), ())),
                               preferred_element_type=jnp.float32)


def _sigmoid(t):
    return 1.0 / (1.0 + jnp.exp(-t))


def _moe_body(x_ref, Wg_ref, We1_ref, be1_ref, Weh_ref, beh_ref,
              Weo_ref, beo_ref, Ws1_ref, bs1_ref, Wsh_ref, bsh_ref,
              Wso_ref, bso_ref, o_ref):
    x = x_ref[...]

    logits = _dot_t(x, Wg_ref[...])
    idx = jax.lax.broadcasted_iota(jnp.int32, logits.shape, 1)
    m1 = jnp.max(logits, axis=-1, keepdims=True)
    i1 = jnp.min(jnp.where(logits == m1, idx, E), axis=-1, keepdims=True)
    masked = jnp.where(idx == i1, -jnp.inf, logits)
    m2 = jnp.max(masked, axis=-1, keepdims=True)
    i2 = jnp.min(jnp.where(masked == m2, idx, E), axis=-1, keepdims=True)
    z = jnp.exp(m2 - m1)
    w1 = 1.0 / (1.0 + z)
    w2 = z * w1
    w = jnp.where(idx == i1, w1, 0.0) + jnp.where(idx == i2, w2, 0.0)

    acc = jnp.zeros((x.shape[0], O), jnp.float32)
    for e in range(E):
        h = jnp.maximum(_dot_t(x, We1_ref[e]) + be1_ref[e], 0.0) * BN_S
        h = jnp.maximum(_dot_t(h, Weh_ref[e]) + beh_ref[e], 0.0) * BN_S
        o = _sigmoid(_dot_t(h, Weo_ref[e]) + beo_ref[e])
        acc = acc + o * w[:, e:e + 1]

    h = jnp.maximum(_dot_t(x, Ws1_ref[...]) + bs1_ref[...], 0.0) * BN_S
    h = jnp.maximum(_dot_t(h, Wsh_ref[...]) + bsh_ref[...], 0.0) * BN_S
    sf = _sigmoid(_dot_t(h, Wso_ref[...]) + bso_ref[...])

    o_ref[...] = acc + sf


@functools.partial(jax.jit, static_argnames=("interpret",))
def _run(combined, Wg, We1, be1, Weh, beh, Weo, beo,
         Ws1, bs1, Wsh, bsh, Wso, bso, interpret=False):
    n_tiles = N // TN
    full = lambda shape: pl.BlockSpec(shape, lambda i: (0,) * len(shape))
    grid_spec = pl.GridSpec(
        grid=(n_tiles,),
        in_specs=[
            pl.BlockSpec((TN, D), lambda i: (i, 0)),
            full((E, D)),
            full((E, H, D)), full((E, H)),
            full((E, H, H)), full((E, H)),
            full((E, O, H)), full((E, O)),
            full((H, D)), full((1, H)),
            full((H, H)), full((1, H)),
            full((O, H)), full((1, O)),
        ],
        out_specs=pl.BlockSpec((TN, O), lambda i: (i, 0)),
    )
    return pl.pallas_call(
        _moe_body,
        grid_spec=grid_spec,
        out_shape=jax.ShapeDtypeStruct((N, O), jnp.float32),
        compiler_params=pltpu.CompilerParams(
            dimension_semantics=("parallel",),
        ),
        interpret=interpret,
    )(combined, Wg, We1, be1, Weh, beh, Weo, beo,
      Ws1, bs1.reshape(1, H), Wsh, bsh.reshape(1, H), Wso, bso.reshape(1, O))


def kernel(combined, Wg, We1, be1, Weh, beh, Weo, beo,
           Ws1, bs1, Wsh, bsh, Wso, bso):
    return _run(combined, Wg, We1, be1, Weh, beh, Weo, beo,
                Ws1, bs1, Wsh, bsh, Wso, bso)
